# baseline (speedup 1.0000x reference)
import functools

import jax
import jax.numpy as jnp
from jax import lax
from jax.experimental import pallas as pl
from jax.experimental.pallas import tpu as pltpu
from jax.experimental.pallas import tpu_sc as plsc

F, B, L, V, D = 26, 4096, 20, 100000, 16
NC, NS, NL = 2, 16, 16
BCH = 1024
NBC = B // BCH
NBG = BCH // NL
FSPLITS = (2, 6, 8, 10)


def _make_call(fa, nf):
    nfh = nf // 2

    def body(tab_hbm, idx_hbm, out_hbm, tile_v, idxc_v, out_v, spm_i,
             sem_i, sem_t):
        c_ax = lax.axis_index("c")
        s_ax = lax.axis_index("s")

        def i_slice(f_glob, t):
            return idx_hbm.at[pl.ds(f_glob * L, L), pl.ds(t * BCH, BCH)]

        def feat_body(j, carry):
            f_glob = fa + c_ax * nfh + j
            row = (c_ax * nfh + j) * D + s_ax

            @pl.when(s_ax == 0)
            def _():
                pltpu.async_copy(i_slice(f_glob, 0), spm_i.at[0], sem_i)
                pltpu.async_copy(i_slice(f_glob, 1), spm_i.at[1], sem_i)

            pltpu.sync_copy(tab_hbm.at[row], tile_v)

            def chunk_body(t, carry2):
                buf = lax.rem(t, 2)

                @pl.when(s_ax == 0)
                def _():
                    pltpu.make_async_copy(
                        i_slice(f_glob, t), spm_i.at[buf], sem_i).wait()

                plsc.subcore_barrier()
                pltpu.sync_copy(spm_i.at[buf], idxc_v)
                plsc.subcore_barrier()

                @pl.when((s_ax == 0) & (t + 2 < NBC))
                def _():
                    pltpu.async_copy(i_slice(f_glob, t + 2), spm_i.at[buf],
                                     sem_i)

                @plsc.parallel_loop(0, NBG, 1, unroll=2)
                def group_body(g):
                    acc0 = jnp.zeros((NL,), jnp.float32)
                    acc1 = jnp.zeros((NL,), jnp.float32)
                    for l in range(0, L, 2):
                        i0 = idxc_v[l, pl.ds(g * NL, NL)]
                        i1 = idxc_v[l + 1, pl.ds(g * NL, NL)]
                        acc0 = acc0 + plsc.load_gather(tile_v, [i0])
                        acc1 = acc1 + plsc.load_gather(tile_v, [i1])
                    out_v[pl.ds(t * BCH + g * NL, NL)] = acc0 + acc1

                return carry2

            lax.fori_loop(0, NBC, chunk_body, 0)
            pltpu.sync_copy(out_v, out_hbm.at[row])
            return carry

        lax.fori_loop(0, nfh, feat_body, 0)

    return functools.partial(
        pl.kernel,
        out_type=jax.ShapeDtypeStruct((nf * D, B), jnp.float32),
        mesh=plsc.VectorSubcoreMesh(core_axis_name="c", subcore_axis_name="s"),
        scratch_types=[
            pltpu.VMEM((V,), jnp.float32),
            pltpu.VMEM((L, BCH), jnp.int32),
            pltpu.VMEM((B,), jnp.float32),
            pltpu.VMEM_SHARED((2, L, BCH), jnp.int32),
            pltpu.SemaphoreType.DMA,
            pltpu.SemaphoreType.DMA,
        ],
        compiler_params=pltpu.CompilerParams(
            use_tc_tiling_on_sc=False, needs_layout_passes=False),
    )(body)


_calls = []
_foff = 0
for _n in FSPLITS:
    _calls.append((_foff, _n, _make_call(_foff, _n)))
    _foff += _n


def kernel(sparse_indices, dense_feature, tables):
    idx_t = sparse_indices.transpose(0, 2, 1).reshape(F * L, B)
    parts = [
        c(tables[fa:fa + n].transpose(0, 2, 1).reshape(n * D, V), idx_t)
        for fa, n, c in _calls
    ]
    cols = [p.T for p in parts] + [dense_feature]
    return jnp.concatenate(cols, axis=1)


# speedup vs baseline: 1.0174x; 1.0174x over previous
import functools

import jax
import jax.numpy as jnp
from jax import lax
from jax.experimental import pallas as pl
from jax.experimental.pallas import tpu as pltpu
from jax.experimental.pallas import tpu_sc as plsc

F, B, L, V, D = 26, 4096, 20, 100000, 16
NC, NS, NL = 2, 16, 16
BCH = 1024
NBC = B // BCH
NBG = BCH // NL
FSPLITS = (2, 6, 8, 8, 2)


def _make_call(fa, nf):
    nfh = nf // 2

    def body(tab_hbm, idx_hbm, out_hbm, tile_v, idxc_v, out_v, spm_i,
             sem_i, sem_t):
        c_ax = lax.axis_index("c")
        s_ax = lax.axis_index("s")

        def i_slice(f_glob, t):
            return idx_hbm.at[pl.ds(f_glob * L, L), pl.ds(t * BCH, BCH)]

        def feat_body(j, carry):
            f_glob = fa + c_ax * nfh + j
            row = (c_ax * nfh + j) * D + s_ax

            @pl.when(s_ax == 0)
            def _():
                pltpu.async_copy(i_slice(f_glob, 0), spm_i.at[0], sem_i)
                pltpu.async_copy(i_slice(f_glob, 1), spm_i.at[1], sem_i)

            pltpu.sync_copy(tab_hbm.at[row], tile_v)

            def chunk_body(t, carry2):
                buf = lax.rem(t, 2)

                @pl.when(s_ax == 0)
                def _():
                    pltpu.make_async_copy(
                        i_slice(f_glob, t), spm_i.at[buf], sem_i).wait()

                plsc.subcore_barrier()
                pltpu.sync_copy(spm_i.at[buf], idxc_v)
                plsc.subcore_barrier()

                @pl.when((s_ax == 0) & (t + 2 < NBC))
                def _():
                    pltpu.async_copy(i_slice(f_glob, t + 2), spm_i.at[buf],
                                     sem_i)

                @plsc.parallel_loop(0, NBG, 1, unroll=2)
                def group_body(g):
                    acc0 = jnp.zeros((NL,), jnp.float32)
                    acc1 = jnp.zeros((NL,), jnp.float32)
                    for l in range(0, L, 2):
                        i0 = idxc_v[l, pl.ds(g * NL, NL)]
                        i1 = idxc_v[l + 1, pl.ds(g * NL, NL)]
                        acc0 = acc0 + plsc.load_gather(tile_v, [i0])
                        acc1 = acc1 + plsc.load_gather(tile_v, [i1])
                    out_v[pl.ds(t * BCH + g * NL, NL)] = acc0 + acc1

                return carry2

            lax.fori_loop(0, NBC, chunk_body, 0)
            pltpu.sync_copy(out_v, out_hbm.at[row])
            return carry

        lax.fori_loop(0, nfh, feat_body, 0)

    return functools.partial(
        pl.kernel,
        out_type=jax.ShapeDtypeStruct((nf * D, B), jnp.float32),
        mesh=plsc.VectorSubcoreMesh(core_axis_name="c", subcore_axis_name="s"),
        scratch_types=[
            pltpu.VMEM((V,), jnp.float32),
            pltpu.VMEM((L, BCH), jnp.int32),
            pltpu.VMEM((B,), jnp.float32),
            pltpu.VMEM_SHARED((2, L, BCH), jnp.int32),
            pltpu.SemaphoreType.DMA,
            pltpu.SemaphoreType.DMA,
        ],
        compiler_params=pltpu.CompilerParams(
            use_tc_tiling_on_sc=False, needs_layout_passes=False),
    )(body)


_calls = []
_foff = 0
for _n in FSPLITS:
    _calls.append((_foff, _n, _make_call(_foff, _n)))
    _foff += _n


def kernel(sparse_indices, dense_feature, tables):
    idx_t = sparse_indices.transpose(0, 2, 1).reshape(F * L, B)
    parts = [
        c(tables[fa:fa + n].transpose(0, 2, 1).reshape(n * D, V), idx_t)
        for fa, n, c in _calls
    ]
    cols = [p.T for p in parts] + [dense_feature]
    return jnp.concatenate(cols, axis=1)
